# Initial kernel scaffold; baseline (speedup 1.0000x reference)
import jax
import jax.numpy as jnp
from jax.experimental import pallas as pl


def kernel(x, edge_index, adj_values, adj2_index, adj2_values, W1, W_out, b_out):
    raise NotImplementedError("write your pallas kernel here")


# speedup vs baseline: 10.1474x; 10.1474x over previous
import functools

import jax
import jax.numpy as jnp
from jax import lax
from jax.experimental import pallas as pl
from jax.experimental.pallas import tpu as pltpu
from jax.experimental.pallas import tpu_sc as plsc

_N = 10000
_IN = 128
_HID = 128
_OUT = 64
_E1 = 320000
_E2 = 640000

_NC = 2
_NS = 16
_NW = _NC * _NS
_K = 80
_C1 = _E1 // (_NW * _K)
_C2 = _E2 // (_NW * _K)
_RPT = 624
_TAIL = _N - _NS * _RPT


def _dense_body(x_ref, w1_ref, wout_ref, b_ref, dense_ref, t1_ref, t2_ref):
    dims = (((1,), (1,)), ((), ()))
    h0 = lax.dot_general(x_ref[...], w1_ref[...], dims,
                         preferred_element_type=jnp.float32)
    wa = wout_ref[:, 0:_HID]
    wb = wout_ref[:, _HID:2 * _HID]
    wc = wout_ref[:, 2 * _HID:3 * _HID]
    dense_ref[...] = lax.dot_general(h0, wa, dims,
                                     preferred_element_type=jnp.float32) + b_ref[0:1, :]
    t1_ref[...] = lax.dot_general(h0, wb, dims, preferred_element_type=jnp.float32)
    t2_ref[...] = lax.dot_general(h0, wc, dims, preferred_element_type=jnp.float32)


def _combine_body(dense_ref, p_ref, out_ref):
    out_ref[...] = dense_ref[...] + p_ref[0] + p_ref[1]


def _sc_body(src1h, dst1h, val1h, src2h, dst2h, val2h, t1h, t2h, zh, outh,
             acc, sb, db, vb, rows, sem):
    c = lax.axis_index("c")
    s = lax.axis_index("s")
    wid = c * _NS + s

    pltpu.sync_copy(zh.at[pl.ds(s * _RPT, _RPT)], acc.at[pl.ds(s * _RPT, _RPT)])

    @pl.when(s == 0)
    def _():
        pltpu.sync_copy(zh.at[pl.ds(_NS * _RPT, _TAIL)],
                        acc.at[pl.ds(_NS * _RPT, _TAIL)])

    plsc.subcore_barrier()

    def run_list(nch, srch, dsth, valh, th):
        pltpu.sync_copy(srch.at[wid], sb.at[pl.ds(0, nch)])
        pltpu.sync_copy(dsth.at[wid], db.at[pl.ds(0, nch)])
        pltpu.sync_copy(valh.at[wid], vb.at[pl.ds(0, nch)])

        def body(k, carry):
            pltpu.async_copy(th.at[sb.at[k]], rows, sem).wait()
            for g in range(_K // 16):
                vv = vb[k, pl.ds(g * 16, 16)]
                for jj in range(16):
                    j = g * 16 + jj
                    v = vv[jj]
                    for q in range(_OUT // 16):
                        sl = pl.ds(q * 16, 16)
                        rows[j, sl] = rows[j, sl] * v
            pltpu.sync_copy(rows, acc.at[db.at[k]], add=True)
            return carry
        lax.fori_loop(0, nch, body, 0)

    run_list(_C1, src1h, dst1h, val1h, t1h)
    run_list(_C2, src2h, dst2h, val2h, t2h)
    plsc.subcore_barrier()

    pltpu.sync_copy(acc.at[pl.ds(s * _RPT, _RPT)],
                    outh.at[c, pl.ds(s * _RPT, _RPT)])

    @pl.when(s == 0)
    def _():
        pltpu.sync_copy(acc.at[pl.ds(_NS * _RPT, _TAIL)],
                        outh.at[c, pl.ds(_NS * _RPT, _TAIL)])


def kernel(x, edge_index, adj_values, adj2_index, adj2_values, W1, W_out, b_out):
    f32 = jnp.float32

    rblk = 2000
    grid = (_N // rblk,)
    dense, t1, t2 = pl.pallas_call(
        _dense_body,
        grid=grid,
        in_specs=[
            pl.BlockSpec((rblk, _IN), lambda i: (i, 0)),
            pl.BlockSpec((_HID, _IN), lambda i: (0, 0)),
            pl.BlockSpec((_OUT, 3 * _HID), lambda i: (0, 0)),
            pl.BlockSpec((8, _OUT), lambda i: (0, 0)),
        ],
        out_specs=[
            pl.BlockSpec((rblk, _OUT), lambda i: (i, 0)),
            pl.BlockSpec((rblk, _OUT), lambda i: (i, 0)),
            pl.BlockSpec((rblk, _OUT), lambda i: (i, 0)),
        ],
        out_shape=[
            jax.ShapeDtypeStruct((_N, _OUT), f32),
            jax.ShapeDtypeStruct((_N, _OUT), f32),
            jax.ShapeDtypeStruct((_N, _OUT), f32),
        ],
    )(x, W1, W_out, jnp.broadcast_to(b_out, (8, _OUT)))

    dst1 = edge_index[0].reshape(_NW, _C1, _K)
    src1 = edge_index[1].reshape(_NW, _C1, _K)
    val1 = adj_values.reshape(_NW, _C1, _K)
    dst2 = adj2_index[0].reshape(_NW, _C2, _K)
    src2 = adj2_index[1].reshape(_NW, _C2, _K)
    val2 = adj2_values.reshape(_NW, _C2, _K)
    zeros = jnp.zeros((_N, _OUT), f32)

    mesh = plsc.VectorSubcoreMesh(core_axis_name="c", subcore_axis_name="s",
                                  num_cores=_NC, num_subcores=_NS)
    partial = pl.kernel(
        _sc_body,
        jax.ShapeDtypeStruct((_NC, _N, _OUT), f32),
        mesh=mesh,
        compiler_params=pltpu.CompilerParams(use_tc_tiling_on_sc=False),
        scratch_types=[
            pltpu.VMEM_SHARED((_N, _OUT), f32),
            pltpu.VMEM((_C2, _K), jnp.int32),
            pltpu.VMEM((_C2, _K), jnp.int32),
            pltpu.VMEM((_C2, _K), f32),
            pltpu.VMEM((_K, _OUT), f32),
            pltpu.SemaphoreType.DMA,
        ],
    )(src1, dst1, val1, src2, dst2, val2, t1, t2, zeros)

    out = pl.pallas_call(
        _combine_body,
        grid=grid,
        in_specs=[
            pl.BlockSpec((rblk, _OUT), lambda i: (i, 0)),
            pl.BlockSpec((_NC, rblk, _OUT), lambda i: (0, i, 0)),
        ],
        out_specs=pl.BlockSpec((rblk, _OUT), lambda i: (i, 0)),
        out_shape=jax.ShapeDtypeStruct((_N, _OUT), f32),
    )(dense, partial)
    return out


# speedup vs baseline: 14.2531x; 1.4046x over previous
import functools

import jax
import jax.numpy as jnp
from jax import lax
from jax.experimental import pallas as pl
from jax.experimental.pallas import tpu as pltpu
from jax.experimental.pallas import tpu_sc as plsc

_N = 10000
_IN = 128
_HID = 128
_OUT = 64
_E1 = 320000
_E2 = 640000

_NC = 2
_NS = 16
_NW = _NC * _NS
_K = 80
_C1 = _E1 // (_NW * _K)
_C2 = _E2 // (_NW * _K)
_RPT = 624
_TAIL = _N - _NS * _RPT


def _dense_body(x_ref, w1_ref, wout_ref, b_ref, dense_ref, t1_ref, t2_ref):
    dims = (((1,), (1,)), ((), ()))
    h0 = lax.dot_general(x_ref[...], w1_ref[...], dims,
                         preferred_element_type=jnp.float32)
    wa = wout_ref[:, 0:_HID]
    wb = wout_ref[:, _HID:2 * _HID]
    wc = wout_ref[:, 2 * _HID:3 * _HID]
    dense_ref[...] = lax.dot_general(h0, wa, dims,
                                     preferred_element_type=jnp.float32) + b_ref[0:1, :]
    t1_ref[...] = lax.dot_general(h0, wb, dims, preferred_element_type=jnp.float32)
    t2_ref[...] = lax.dot_general(h0, wc, dims, preferred_element_type=jnp.float32)


def _combine_body(dense_ref, p_ref, out_ref):
    out_ref[...] = dense_ref[...] + p_ref[0] + p_ref[1]


def _sc_body(src1h, dst1h, val1h, src2h, dst2h, val2h, t1h, t2h, zh, outh,
             acc, sb, db, vb, rows_a, rows_b, sem_a, sem_b):
    c = lax.axis_index("c")
    s = lax.axis_index("s")
    wid = c * _NS + s

    pltpu.sync_copy(zh.at[pl.ds(s * _RPT, _RPT)], acc.at[pl.ds(s * _RPT, _RPT)])

    @pl.when(s == 0)
    def _():
        pltpu.sync_copy(zh.at[pl.ds(_NS * _RPT, _TAIL)],
                        acc.at[pl.ds(_NS * _RPT, _TAIL)])

    plsc.subcore_barrier()

    def mult_scatter(rows, k):
        for g in range(_K // 16):
            vv = vb[k, pl.ds(g * 16, 16)]
            for jj in range(16):
                j = g * 16 + jj
                v = vv[jj]
                for q in range(_OUT // 16):
                    sl = pl.ds(q * 16, 16)
                    rows[j, sl] = rows[j, sl] * v
        pltpu.sync_copy(rows, acc.at[db.at[k]], add=True)

    def run_list(nch, srch, dsth, valh, th):
        pltpu.sync_copy(srch.at[wid], sb.at[pl.ds(0, nch)])
        pltpu.sync_copy(dsth.at[wid], db.at[pl.ds(0, nch)])
        pltpu.sync_copy(valh.at[wid], vb.at[pl.ds(0, nch)])

        base = nch % 2
        if base:
            pltpu.async_copy(th.at[sb.at[0]], rows_a, sem_a).wait()
            mult_scatter(rows_a, 0)
        pltpu.async_copy(th.at[sb.at[base]], rows_a, sem_a)

        def body(i, carry):
            a = base + 2 * i
            b = a + 1
            nxt = jnp.minimum(a + 2, nch - 1)
            pltpu.make_async_copy(th.at[sb.at[a]], rows_a, sem_a).wait()
            pltpu.async_copy(th.at[sb.at[b]], rows_b, sem_b)
            mult_scatter(rows_a, a)
            pltpu.make_async_copy(th.at[sb.at[b]], rows_b, sem_b).wait()
            pltpu.async_copy(th.at[sb.at[nxt]], rows_a, sem_a)
            mult_scatter(rows_b, b)
            return carry
        lax.fori_loop(0, (nch - base) // 2, body, 0)
        pltpu.make_async_copy(th.at[sb.at[0]], rows_a, sem_a).wait()

    run_list(_C1, src1h, dst1h, val1h, t1h)
    run_list(_C2, src2h, dst2h, val2h, t2h)
    plsc.subcore_barrier()

    pltpu.sync_copy(acc.at[pl.ds(s * _RPT, _RPT)],
                    outh.at[c, pl.ds(s * _RPT, _RPT)])

    @pl.when(s == 0)
    def _():
        pltpu.sync_copy(acc.at[pl.ds(_NS * _RPT, _TAIL)],
                        outh.at[c, pl.ds(_NS * _RPT, _TAIL)])


def kernel(x, edge_index, adj_values, adj2_index, adj2_values, W1, W_out, b_out):
    f32 = jnp.float32

    rblk = 2000
    grid = (_N // rblk,)
    dense, t1, t2 = pl.pallas_call(
        _dense_body,
        grid=grid,
        in_specs=[
            pl.BlockSpec((rblk, _IN), lambda i: (i, 0)),
            pl.BlockSpec((_HID, _IN), lambda i: (0, 0)),
            pl.BlockSpec((_OUT, 3 * _HID), lambda i: (0, 0)),
            pl.BlockSpec((8, _OUT), lambda i: (0, 0)),
        ],
        out_specs=[
            pl.BlockSpec((rblk, _OUT), lambda i: (i, 0)),
            pl.BlockSpec((rblk, _OUT), lambda i: (i, 0)),
            pl.BlockSpec((rblk, _OUT), lambda i: (i, 0)),
        ],
        out_shape=[
            jax.ShapeDtypeStruct((_N, _OUT), f32),
            jax.ShapeDtypeStruct((_N, _OUT), f32),
            jax.ShapeDtypeStruct((_N, _OUT), f32),
        ],
    )(x, W1, W_out, jnp.broadcast_to(b_out, (8, _OUT)))

    dst1 = edge_index[0].reshape(_NW, _C1, _K)
    src1 = edge_index[1].reshape(_NW, _C1, _K)
    val1 = adj_values.reshape(_NW, _C1, _K)
    dst2 = adj2_index[0].reshape(_NW, _C2, _K)
    src2 = adj2_index[1].reshape(_NW, _C2, _K)
    val2 = adj2_values.reshape(_NW, _C2, _K)
    zeros = jnp.zeros((_N, _OUT), f32)

    mesh = plsc.VectorSubcoreMesh(core_axis_name="c", subcore_axis_name="s",
                                  num_cores=_NC, num_subcores=_NS)
    partial = pl.kernel(
        _sc_body,
        jax.ShapeDtypeStruct((_NC, _N, _OUT), f32),
        mesh=mesh,
        compiler_params=pltpu.CompilerParams(use_tc_tiling_on_sc=False),
        scratch_types=[
            pltpu.VMEM_SHARED((_N, _OUT), f32),
            pltpu.VMEM((_C2, _K), jnp.int32),
            pltpu.VMEM((_C2, _K), jnp.int32),
            pltpu.VMEM((_C2, _K), f32),
            pltpu.VMEM((_K, _OUT), f32),
            pltpu.VMEM((_K, _OUT), f32),
            pltpu.SemaphoreType.DMA,
            pltpu.SemaphoreType.DMA,
        ],
    )(src1, dst1, val1, src2, dst2, val2, t1, t2, zeros)

    out = pl.pallas_call(
        _combine_body,
        grid=grid,
        in_specs=[
            pl.BlockSpec((rblk, _OUT), lambda i: (i, 0)),
            pl.BlockSpec((_NC, rblk, _OUT), lambda i: (0, i, 0)),
        ],
        out_specs=pl.BlockSpec((rblk, _OUT), lambda i: (i, 0)),
        out_shape=jax.ShapeDtypeStruct((_N, _OUT), f32),
    )(dense, partial)
    return out
